# baseline (speedup 1.0000x reference)
import functools

import jax
import jax.numpy as jnp
from jax import lax
from jax.experimental import pallas as pl
from jax.experimental.pallas import tpu as pltpu
from jax.experimental.pallas import tpu_sc as plsc

N = 10000
E = 320000
D = 128

NC = 2
NS = 16
NW = NC * NS

NP = 10240
PAD_NODE = N
CH = 128
EP = 327680
CPT = EP // (NW * CH)
ROWS_PT = NP // NS

_mesh = plsc.VectorSubcoreMesh(core_axis_name="c", subcore_axis_name="s",
                               num_cores=NC, num_subcores=NS)


@functools.partial(
    pl.kernel,
    out_type=jax.ShapeDtypeStruct((NC * 2 * NP,), jnp.float32),
    mesh=_mesh,
    scratch_types=[
        pltpu.VMEM((CPT, CH), jnp.int32),
        pltpu.VMEM((CPT, CH), jnp.int32),
        pltpu.VMEM((CH,), jnp.float32),
        pltpu.VMEM_SHARED((NP,), jnp.float32),
        pltpu.VMEM_SHARED((NP,), jnp.float32),
        pltpu.SemaphoreType.DMA,
        pltpu.SemaphoreType.DMA,
    ],
)
def _deg_kernel(src_hbm, dst_hbm, ones_hbm, zeros_hbm,
                out, src_v, dst_v, ones_v, hist_s, hist_d, sem_s, sem_d):
  c = lax.axis_index("c")
  s = lax.axis_index("s")
  t = c * NS + s

  rbase = s * ROWS_PT
  pltpu.sync_copy(zeros_hbm.at[pl.ds(rbase, ROWS_PT)],
                  hist_s.at[pl.ds(rbase, ROWS_PT)])
  pltpu.sync_copy(zeros_hbm.at[pl.ds(rbase, ROWS_PT)],
                  hist_d.at[pl.ds(rbase, ROWS_PT)])
  pltpu.sync_copy(ones_hbm, ones_v)
  pltpu.sync_copy(src_hbm.at[t], src_v)
  pltpu.sync_copy(dst_hbm.at[t], dst_v)
  plsc.subcore_barrier()

  def body(j, carry):
    pltpu.sync_copy(ones_v, hist_s.at[src_v.at[j]], add=True)
    pltpu.sync_copy(ones_v, hist_d.at[dst_v.at[j]], add=True)
    return carry
  lax.fori_loop(0, CPT, body, 0)

  plsc.subcore_barrier()
  pltpu.sync_copy(hist_s.at[pl.ds(rbase, ROWS_PT)],
                  out.at[pl.ds(2 * c * NP + rbase, ROWS_PT)])
  pltpu.sync_copy(hist_d.at[pl.ds(rbase, ROWS_PT)],
                  out.at[pl.ds((2 * c + 1) * NP + rbase, ROWS_PT)])


@functools.partial(
    pl.kernel,
    out_type=jax.ShapeDtypeStruct((NC, NP, D), jnp.float32),
    mesh=_mesh,
    scratch_types=[
        pltpu.VMEM((CPT // 2, CH), jnp.int32),
        pltpu.VMEM((CPT // 2, CH), jnp.int32),
        pltpu.VMEM((CH, D), jnp.float32),
        pltpu.VMEM((CH, D), jnp.float32),
        pltpu.VMEM_SHARED((NP, D), jnp.float32),
        pltpu.SemaphoreType.DMA,
        pltpu.SemaphoreType.DMA,
        pltpu.SemaphoreType.DMA,
        pltpu.SemaphoreType.DMA,
    ],
)
def _agg_kernel(h_hbm, src_hbm, dst_hbm, zeros_hbm,
                out, src_v, dst_v, gbuf0, gbuf1, acc,
                semg0, semg1, sems0, sems1):
  c = lax.axis_index("c")
  s = lax.axis_index("s")
  t = c * NS + s
  HC = CPT // 2

  rbase = s * ROWS_PT
  pltpu.sync_copy(zeros_hbm.at[pl.ds(rbase, ROWS_PT)],
                  acc.at[pl.ds(rbase, ROWS_PT)])
  plsc.subcore_barrier()

  def gdrain(ch, buf, sem):
    pltpu.make_async_copy(h_hbm.at[src_v.at[ch]], buf, sem).wait()

  def sdrain(ch, buf, sem):
    pltpu.make_async_copy(buf, acc.at[dst_v.at[ch]], sem).wait()

  def run_half(h):
    pltpu.sync_copy(src_hbm.at[t, pl.ds(h * HC, HC)], src_v)
    pltpu.sync_copy(dst_hbm.at[t, pl.ds(h * HC, HC)], dst_v)
    pltpu.async_copy(h_hbm.at[src_v.at[0]], gbuf0, semg0)

    def body(j2, carry):
      a = 2 * j2
      b = a + 1

      @pl.when(j2 > 0)
      def _():
        sdrain(a - 1, gbuf1, sems1)
      pltpu.async_copy(h_hbm.at[src_v.at[b]], gbuf1, semg1)
      gdrain(a, gbuf0, semg0)
      pltpu.async_copy(gbuf0, acc.at[dst_v.at[a]], sems0, add=True)
      gdrain(b, gbuf1, semg1)
      pltpu.async_copy(gbuf1, acc.at[dst_v.at[b]], sems1, add=True)

      @pl.when(j2 + 1 < HC // 2)
      def _():
        sdrain(a, gbuf0, sems0)
        pltpu.async_copy(h_hbm.at[src_v.at[a + 2]], gbuf0, semg0)
      return carry
    lax.fori_loop(0, HC // 2, body, 0)
    sdrain(HC - 2, gbuf0, sems0)
    sdrain(HC - 1, gbuf1, sems1)

  run_half(0)
  run_half(1)

  plsc.subcore_barrier()
  pltpu.sync_copy(acc.at[pl.ds(rbase, ROWS_PT)],
                  out.at[c, pl.ds(rbase, ROWS_PT)])


def _prep_body(x_ref, s0_ref, s1_ref, d0_ref, d1_ref,
               xs_ref, nsrc_ref, ndst_ref):
  deg_s = s0_ref[...] + s1_ref[...]
  deg_d = d0_ref[...] + d1_ref[...]
  nsrc = lax.rsqrt(jnp.maximum(deg_s, 1.0))
  ndst = lax.rsqrt(jnp.maximum(deg_d, 1.0))
  nsrc_ref[...] = nsrc
  ndst_ref[...] = ndst
  xs_ref[0:N] = x_ref[...] * nsrc[0:N]
  xs_ref[N:NP] = jnp.zeros((NP - N, D), jnp.float32)


def _layer_a_body(p_ref, ndst_ref, w_ref, b_ref, nsrc_ref, out_ref):
  agg = (p_ref[0] + p_ref[1]) * ndst_ref[...]
  h = jnp.dot(agg, w_ref[...], preferred_element_type=jnp.float32)
  h = jnp.maximum(h + b_ref[...], 0.0)
  out_ref[...] = h * nsrc_ref[...]


def _layer_b_body(p_ref, ndst_ref, w_ref, b_ref, fcw_ref, fcb_ref, out_ref):
  agg = (p_ref[0, 0:N] + p_ref[1, 0:N]) * ndst_ref[0:N]
  h = jnp.dot(agg, w_ref[...], preferred_element_type=jnp.float32)
  h = jnp.maximum(h + b_ref[...], 0.0)
  hg = jnp.sum(h, axis=0, keepdims=True) * (1.0 / N)
  logit = jnp.dot(hg, fcw_ref[...], preferred_element_type=jnp.float32)
  out_ref[...] = 1.0 / (1.0 + jnp.exp(-(logit + fcb_ref[...])))


def kernel(x, edge_index, W1, b1, W2, b2, fc_W, fc_b):
  pad_idx = PAD_NODE + (jnp.arange(EP - E, dtype=jnp.int32) % (NP - N))
  ei = jnp.concatenate([edge_index, jnp.tile(pad_idx, (2, 1))], axis=1)
  src = ei[0].reshape(NW, CPT, CH)
  dst = ei[1].reshape(NW, CPT, CH)

  ones1 = jnp.ones((CH,), jnp.float32)
  zeros1 = jnp.zeros((NP,), jnp.float32)
  zerosD = jnp.zeros((NP, D), jnp.float32)

  degflat = _deg_kernel(src, dst, ones1, zeros1)
  d_s0 = degflat[0 * NP:1 * NP].reshape(NP, 1)
  d_d0 = degflat[1 * NP:2 * NP].reshape(NP, 1)
  d_s1 = degflat[2 * NP:3 * NP].reshape(NP, 1)
  d_d1 = degflat[3 * NP:4 * NP].reshape(NP, 1)

  xs, nsrc, ndst = pl.pallas_call(
      _prep_body,
      out_shape=(
          jax.ShapeDtypeStruct((NP, D), jnp.float32),
          jax.ShapeDtypeStruct((NP, 1), jnp.float32),
          jax.ShapeDtypeStruct((NP, 1), jnp.float32),
      ),
  )(x, d_s0, d_s1, d_d0, d_d1)

  p1 = _agg_kernel(xs, src, dst, zerosD)

  h1s = pl.pallas_call(
      _layer_a_body,
      out_shape=jax.ShapeDtypeStruct((NP, D), jnp.float32),
  )(p1, ndst, W1, b1.reshape(1, D), nsrc)

  p2 = _agg_kernel(h1s, src, dst, zerosD)

  out = pl.pallas_call(
      _layer_b_body,
      out_shape=jax.ShapeDtypeStruct((1, 1), jnp.float32),
  )(p2, ndst, W2, b2.reshape(1, D), fc_W, fc_b.reshape(1, 1))
  return out


# speedup vs baseline: 1.2241x; 1.2241x over previous
import functools

import jax
import jax.numpy as jnp
from jax import lax
from jax.experimental import pallas as pl
from jax.experimental.pallas import tpu as pltpu
from jax.experimental.pallas import tpu_sc as plsc

N = 10000
E = 320000
D = 128

NC = 2
NS = 16
NW = NC * NS

NP = 10240
PAD_NODE = N
CH = 128
EP = 327680
CPT = EP // (NW * CH)
ROWS_PT = NP // NS

_mesh = plsc.VectorSubcoreMesh(core_axis_name="c", subcore_axis_name="s",
                               num_cores=NC, num_subcores=NS)


@functools.partial(
    pl.kernel,
    out_type=jax.ShapeDtypeStruct((NC * 2 * NP,), jnp.float32),
    mesh=_mesh,
    scratch_types=[
        pltpu.VMEM((CPT, CH), jnp.int32),
        pltpu.VMEM((CPT, CH), jnp.int32),
        pltpu.VMEM((CH,), jnp.float32),
        pltpu.VMEM_SHARED((NP,), jnp.float32),
        pltpu.VMEM_SHARED((NP,), jnp.float32),
        pltpu.SemaphoreType.DMA,
        pltpu.SemaphoreType.DMA,
    ],
)
def _deg_kernel(src_hbm, dst_hbm, ones_hbm, zeros_hbm,
                out, src_v, dst_v, ones_v, hist_s, hist_d, sem_s, sem_d):
  c = lax.axis_index("c")
  s = lax.axis_index("s")
  t = c * NS + s

  rbase = s * ROWS_PT
  pltpu.sync_copy(zeros_hbm.at[pl.ds(rbase, ROWS_PT)],
                  hist_s.at[pl.ds(rbase, ROWS_PT)])
  pltpu.sync_copy(zeros_hbm.at[pl.ds(rbase, ROWS_PT)],
                  hist_d.at[pl.ds(rbase, ROWS_PT)])
  pltpu.sync_copy(ones_hbm, ones_v)
  pltpu.sync_copy(src_hbm.at[t], src_v)
  pltpu.sync_copy(dst_hbm.at[t], dst_v)
  plsc.subcore_barrier()

  def body(j, carry):
    pltpu.sync_copy(ones_v, hist_s.at[src_v.at[j]], add=True)
    pltpu.sync_copy(ones_v, hist_d.at[dst_v.at[j]], add=True)
    return carry
  lax.fori_loop(0, CPT, body, 0)

  plsc.subcore_barrier()
  pltpu.sync_copy(hist_s.at[pl.ds(rbase, ROWS_PT)],
                  out.at[pl.ds(2 * c * NP + rbase, ROWS_PT)])
  pltpu.sync_copy(hist_d.at[pl.ds(rbase, ROWS_PT)],
                  out.at[pl.ds((2 * c + 1) * NP + rbase, ROWS_PT)])


@functools.partial(
    pl.kernel,
    out_type=jax.ShapeDtypeStruct((NC, NP, D), jnp.float32),
    mesh=_mesh,
    scratch_types=[
        pltpu.VMEM((CPT // 2, CH), jnp.int32),
        pltpu.VMEM((CPT // 2, CH), jnp.int32),
        pltpu.VMEM((CH, D), jnp.float32),
        pltpu.VMEM((CH, D), jnp.float32),
        pltpu.VMEM_SHARED((NP, D), jnp.float32),
        pltpu.SemaphoreType.DMA,
        pltpu.SemaphoreType.DMA,
        pltpu.SemaphoreType.DMA,
        pltpu.SemaphoreType.DMA,
    ],
)
def _agg_kernel(h_hbm, src_hbm, dst_hbm, zeros_hbm,
                out, src_v, dst_v, gbuf0, gbuf1, acc,
                semg0, semg1, sems0, sems1):
  c = lax.axis_index("c")
  s = lax.axis_index("s")
  t = c * NS + s
  HC = CPT // 2

  rbase = s * ROWS_PT
  pltpu.sync_copy(zeros_hbm.at[pl.ds(rbase, ROWS_PT)],
                  acc.at[pl.ds(rbase, ROWS_PT)])
  plsc.subcore_barrier()

  def run_half(h):
    pltpu.sync_copy(src_hbm.at[t, pl.ds(h * HC, HC)], src_v)
    pltpu.sync_copy(dst_hbm.at[t, pl.ds(h * HC, HC)], dst_v)
    pltpu.async_copy(h_hbm.at[src_v.at[0]], gbuf0, semg0)

    def body(j2, carry):
      a = 2 * j2
      b = a + 1
      pltpu.async_copy(h_hbm.at[src_v.at[b]], gbuf1, semg1)
      pltpu.make_async_copy(h_hbm.at[src_v.at[a]], gbuf0, semg0).wait()
      pltpu.sync_copy(gbuf0, acc.at[dst_v.at[a]], add=True)

      @pl.when(j2 + 1 < HC // 2)
      def _():
        pltpu.async_copy(h_hbm.at[src_v.at[a + 2]], gbuf0, semg0)

      pltpu.make_async_copy(h_hbm.at[src_v.at[b]], gbuf1, semg1).wait()
      pltpu.sync_copy(gbuf1, acc.at[dst_v.at[b]], add=True)
      return carry
    lax.fori_loop(0, HC // 2, body, 0)

  run_half(0)
  run_half(1)

  plsc.subcore_barrier()
  pltpu.sync_copy(acc.at[pl.ds(rbase, ROWS_PT)],
                  out.at[c, pl.ds(rbase, ROWS_PT)])


def _prep_body(x_ref, s0_ref, s1_ref, d0_ref, d1_ref,
               xs_ref, nsrc_ref, ndst_ref):
  deg_s = s0_ref[...] + s1_ref[...]
  deg_d = d0_ref[...] + d1_ref[...]
  nsrc = lax.rsqrt(jnp.maximum(deg_s, 1.0))
  ndst = lax.rsqrt(jnp.maximum(deg_d, 1.0))
  nsrc_ref[...] = nsrc
  ndst_ref[...] = ndst
  xs_ref[0:N] = x_ref[...] * nsrc[0:N]
  xs_ref[N:NP] = jnp.zeros((NP - N, D), jnp.float32)


def _layer_a_body(p_ref, ndst_ref, w_ref, b_ref, nsrc_ref, out_ref):
  agg = (p_ref[0] + p_ref[1]) * ndst_ref[...]
  h = jnp.dot(agg, w_ref[...], preferred_element_type=jnp.float32)
  h = jnp.maximum(h + b_ref[...], 0.0)
  out_ref[...] = h * nsrc_ref[...]


def _layer_b_body(p_ref, ndst_ref, w_ref, b_ref, fcw_ref, fcb_ref, out_ref):
  agg = (p_ref[0, 0:N] + p_ref[1, 0:N]) * ndst_ref[0:N]
  h = jnp.dot(agg, w_ref[...], preferred_element_type=jnp.float32)
  h = jnp.maximum(h + b_ref[...], 0.0)
  hg = jnp.sum(h, axis=0, keepdims=True) * (1.0 / N)
  logit = jnp.dot(hg, fcw_ref[...], preferred_element_type=jnp.float32)
  out_ref[...] = 1.0 / (1.0 + jnp.exp(-(logit + fcb_ref[...])))


def kernel(x, edge_index, W1, b1, W2, b2, fc_W, fc_b):
  pad_idx = PAD_NODE + (jnp.arange(EP - E, dtype=jnp.int32) % (NP - N))
  ei = jnp.concatenate([edge_index, jnp.tile(pad_idx, (2, 1))], axis=1)
  src = ei[0].reshape(NW, CPT, CH)
  dst = ei[1].reshape(NW, CPT, CH)

  ones1 = jnp.ones((CH,), jnp.float32)
  zeros1 = jnp.zeros((NP,), jnp.float32)
  zerosD = jnp.zeros((NP, D), jnp.float32)

  degflat = _deg_kernel(src, dst, ones1, zeros1)
  d_s0 = degflat[0 * NP:1 * NP].reshape(NP, 1)
  d_d0 = degflat[1 * NP:2 * NP].reshape(NP, 1)
  d_s1 = degflat[2 * NP:3 * NP].reshape(NP, 1)
  d_d1 = degflat[3 * NP:4 * NP].reshape(NP, 1)

  xs, nsrc, ndst = pl.pallas_call(
      _prep_body,
      out_shape=(
          jax.ShapeDtypeStruct((NP, D), jnp.float32),
          jax.ShapeDtypeStruct((NP, 1), jnp.float32),
          jax.ShapeDtypeStruct((NP, 1), jnp.float32),
      ),
  )(x, d_s0, d_s1, d_d0, d_d1)

  p1 = _agg_kernel(xs, src, dst, zerosD)

  h1s = pl.pallas_call(
      _layer_a_body,
      out_shape=jax.ShapeDtypeStruct((NP, D), jnp.float32),
  )(p1, ndst, W1, b1.reshape(1, D), nsrc)

  p2 = _agg_kernel(h1s, src, dst, zerosD)

  out = pl.pallas_call(
      _layer_b_body,
      out_shape=jax.ShapeDtypeStruct((1, 1), jnp.float32),
  )(p2, ndst, W2, b2.reshape(1, D), fc_W, fc_b.reshape(1, 1))
  return out


# speedup vs baseline: 1.2482x; 1.0197x over previous
import functools

import jax
import jax.numpy as jnp
from jax import lax
from jax.experimental import pallas as pl
from jax.experimental.pallas import tpu as pltpu
from jax.experimental.pallas import tpu_sc as plsc

N = 10000
E = 320000
D = 128

NC = 2
NS = 16
NW = NC * NS

NP = 10240
PAD_NODE = N
CH = 128
EP = 327680
CPT = EP // (NW * CH)
ROWS_PT = NP // NS

_mesh = plsc.VectorSubcoreMesh(core_axis_name="c", subcore_axis_name="s",
                               num_cores=NC, num_subcores=NS)


@functools.partial(
    pl.kernel,
    out_type=jax.ShapeDtypeStruct((NC * 2 * NP,), jnp.float32),
    mesh=_mesh,
    scratch_types=[
        pltpu.VMEM((CPT, CH), jnp.int32),
        pltpu.VMEM((CPT, CH), jnp.int32),
        pltpu.VMEM((CH,), jnp.float32),
        pltpu.VMEM_SHARED((NP,), jnp.float32),
        pltpu.VMEM_SHARED((NP,), jnp.float32),
        pltpu.SemaphoreType.DMA,
        pltpu.SemaphoreType.DMA,
    ],
)
def _deg_kernel(src_hbm, dst_hbm, ones_hbm, zeros_hbm,
                out, src_v, dst_v, ones_v, hist_s, hist_d, sem_s, sem_d):
  c = lax.axis_index("c")
  s = lax.axis_index("s")
  t = c * NS + s

  rbase = s * ROWS_PT
  pltpu.sync_copy(zeros_hbm.at[pl.ds(rbase, ROWS_PT)],
                  hist_s.at[pl.ds(rbase, ROWS_PT)])
  pltpu.sync_copy(zeros_hbm.at[pl.ds(rbase, ROWS_PT)],
                  hist_d.at[pl.ds(rbase, ROWS_PT)])
  pltpu.sync_copy(ones_hbm, ones_v)
  pltpu.sync_copy(src_hbm.at[t], src_v)
  pltpu.sync_copy(dst_hbm.at[t], dst_v)
  plsc.subcore_barrier()

  def body(j, carry):
    @pl.when(j > 0)
    def _():
      pltpu.make_async_copy(ones_v, hist_s.at[src_v.at[j - 1]], sem_s).wait()
      pltpu.make_async_copy(ones_v, hist_d.at[dst_v.at[j - 1]], sem_d).wait()
    pltpu.async_copy(ones_v, hist_s.at[src_v.at[j]], sem_s, add=True)
    pltpu.async_copy(ones_v, hist_d.at[dst_v.at[j]], sem_d, add=True)
    return carry
  lax.fori_loop(0, CPT, body, 0)
  pltpu.make_async_copy(ones_v, hist_s.at[src_v.at[CPT - 1]], sem_s).wait()
  pltpu.make_async_copy(ones_v, hist_d.at[dst_v.at[CPT - 1]], sem_d).wait()

  plsc.subcore_barrier()
  pltpu.sync_copy(hist_s.at[pl.ds(rbase, ROWS_PT)],
                  out.at[pl.ds(2 * c * NP + rbase, ROWS_PT)])
  pltpu.sync_copy(hist_d.at[pl.ds(rbase, ROWS_PT)],
                  out.at[pl.ds((2 * c + 1) * NP + rbase, ROWS_PT)])


@functools.partial(
    pl.kernel,
    out_type=jax.ShapeDtypeStruct((NC, NP, D), jnp.float32),
    mesh=_mesh,
    scratch_types=[
        pltpu.VMEM((CPT // 2, CH), jnp.int32),
        pltpu.VMEM((CPT // 2, CH), jnp.int32),
        pltpu.VMEM((CH, D), jnp.float32),
        pltpu.VMEM((CH, D), jnp.float32),
        pltpu.VMEM_SHARED((NP, D), jnp.float32),
        pltpu.SemaphoreType.DMA,
        pltpu.SemaphoreType.DMA,
        pltpu.SemaphoreType.DMA,
        pltpu.SemaphoreType.DMA,
    ],
)
def _agg_kernel(h_hbm, src_hbm, dst_hbm, zeros_hbm,
                out, src_v, dst_v, gbuf0, gbuf1, acc,
                semg0, semg1, sems0, sems1):
  c = lax.axis_index("c")
  s = lax.axis_index("s")
  t = c * NS + s
  HC = CPT // 2

  rbase = s * ROWS_PT
  pltpu.sync_copy(zeros_hbm.at[pl.ds(rbase, ROWS_PT)],
                  acc.at[pl.ds(rbase, ROWS_PT)])
  plsc.subcore_barrier()

  def run_half(h):
    pltpu.sync_copy(src_hbm.at[t, pl.ds(h * HC, HC)], src_v)
    pltpu.sync_copy(dst_hbm.at[t, pl.ds(h * HC, HC)], dst_v)
    pltpu.async_copy(h_hbm.at[src_v.at[0]], gbuf0, semg0)

    def body(j2, carry):
      a = 2 * j2
      b = a + 1
      pltpu.async_copy(h_hbm.at[src_v.at[b]], gbuf1, semg1)
      pltpu.make_async_copy(h_hbm.at[src_v.at[a]], gbuf0, semg0).wait()
      pltpu.sync_copy(gbuf0, acc.at[dst_v.at[a]], add=True)

      @pl.when(j2 + 1 < HC // 2)
      def _():
        pltpu.async_copy(h_hbm.at[src_v.at[a + 2]], gbuf0, semg0)

      pltpu.make_async_copy(h_hbm.at[src_v.at[b]], gbuf1, semg1).wait()
      pltpu.sync_copy(gbuf1, acc.at[dst_v.at[b]], add=True)
      return carry
    lax.fori_loop(0, HC // 2, body, 0)

  run_half(0)
  run_half(1)

  plsc.subcore_barrier()
  pltpu.sync_copy(acc.at[pl.ds(rbase, ROWS_PT)],
                  out.at[c, pl.ds(rbase, ROWS_PT)])


def _prep_body(x_ref, s0_ref, s1_ref, d0_ref, d1_ref,
               xs_ref, nsrc_ref, ndst_ref):
  deg_s = s0_ref[...] + s1_ref[...]
  deg_d = d0_ref[...] + d1_ref[...]
  nsrc = lax.rsqrt(jnp.maximum(deg_s, 1.0))
  ndst = lax.rsqrt(jnp.maximum(deg_d, 1.0))
  nsrc_ref[...] = nsrc
  ndst_ref[...] = ndst
  xs_ref[0:N] = x_ref[...] * nsrc[0:N]
  xs_ref[N:NP] = jnp.zeros((NP - N, D), jnp.float32)


def _layer_a_body(p_ref, ndst_ref, w_ref, b_ref, nsrc_ref, out_ref):
  agg = (p_ref[0] + p_ref[1]) * ndst_ref[...]
  h = jnp.dot(agg, w_ref[...], preferred_element_type=jnp.float32)
  h = jnp.maximum(h + b_ref[...], 0.0)
  out_ref[...] = h * nsrc_ref[...]


def _layer_b_body(p_ref, ndst_ref, w_ref, b_ref, fcw_ref, fcb_ref, out_ref):
  agg = (p_ref[0, 0:N] + p_ref[1, 0:N]) * ndst_ref[0:N]
  h = jnp.dot(agg, w_ref[...], preferred_element_type=jnp.float32)
  h = jnp.maximum(h + b_ref[...], 0.0)
  hg = jnp.sum(h, axis=0, keepdims=True) * (1.0 / N)
  logit = jnp.dot(hg, fcw_ref[...], preferred_element_type=jnp.float32)
  out_ref[...] = 1.0 / (1.0 + jnp.exp(-(logit + fcb_ref[...])))


def kernel(x, edge_index, W1, b1, W2, b2, fc_W, fc_b):
  pad_idx = PAD_NODE + (jnp.arange(EP - E, dtype=jnp.int32) % (NP - N))
  ei = jnp.concatenate([edge_index, jnp.tile(pad_idx, (2, 1))], axis=1)
  src = ei[0].reshape(NW, CPT, CH)
  dst = ei[1].reshape(NW, CPT, CH)

  ones1 = jnp.ones((CH,), jnp.float32)
  zeros1 = jnp.zeros((NP,), jnp.float32)
  zerosD = jnp.zeros((NP, D), jnp.float32)

  degflat = _deg_kernel(src, dst, ones1, zeros1)
  d_s0 = degflat[0 * NP:1 * NP].reshape(NP, 1)
  d_d0 = degflat[1 * NP:2 * NP].reshape(NP, 1)
  d_s1 = degflat[2 * NP:3 * NP].reshape(NP, 1)
  d_d1 = degflat[3 * NP:4 * NP].reshape(NP, 1)

  xs, nsrc, ndst = pl.pallas_call(
      _prep_body,
      out_shape=(
          jax.ShapeDtypeStruct((NP, D), jnp.float32),
          jax.ShapeDtypeStruct((NP, 1), jnp.float32),
          jax.ShapeDtypeStruct((NP, 1), jnp.float32),
      ),
  )(x, d_s0, d_s1, d_d0, d_d1)

  p1 = _agg_kernel(xs, src, dst, zerosD)

  h1s = pl.pallas_call(
      _layer_a_body,
      out_shape=jax.ShapeDtypeStruct((NP, D), jnp.float32),
  )(p1, ndst, W1, b1.reshape(1, D), nsrc)

  p2 = _agg_kernel(h1s, src, dst, zerosD)

  out = pl.pallas_call(
      _layer_b_body,
      out_shape=jax.ShapeDtypeStruct((1, 1), jnp.float32),
  )(p2, ndst, W2, b2.reshape(1, D), fc_W, fc_b.reshape(1, 1))
  return out
